# baseline (speedup 1.0000x reference)
import functools

import jax
import jax.numpy as jnp
from jax.experimental import pallas as pl
from jax.experimental.pallas import tpu as pltpu


def _pipelined_linear(nchunks, ch, h_hbm, wt_ref, b_ref, o_hbm,
                      inbuf, outbuf, insem, outsem):
    def in_copy(i):
        return pltpu.make_async_copy(
            h_hbm.at[pl.ds(i * ch, ch), :],
            inbuf.at[pl.ds(i * ch, ch), :],
            insem.at[i],
        )

    for i in range(nchunks):
        in_copy(i).start()
    for i in range(nchunks):
        in_copy(i).wait()
        rows = pl.ds(i * ch, ch)
        acc = jnp.dot(inbuf[rows, :], wt_ref[:, :],
                      preferred_element_type=jnp.float32)
        outbuf[rows, :] = acc + b_ref[:, :]
    out = pltpu.make_async_copy(outbuf, o_hbm, outsem)
    out.start()
    out.wait()


def kernel(h, edge_index, edge_sh, edge_radial, n_atoms, W, b):
    n, d = h.shape
    ch = 1000
    nchunks = n // ch if (n % ch == 0) else 1
    if n % ch != 0:
        ch = n
    wt = W.T
    b2 = b.reshape(1, d)
    return pl.pallas_call(
        functools.partial(_pipelined_linear, nchunks, ch),
        in_specs=[
            pl.BlockSpec(memory_space=pl.ANY),
            pl.BlockSpec((d, d), lambda: (0, 0)),
            pl.BlockSpec((1, d), lambda: (0, 0)),
        ],
        out_specs=pl.BlockSpec(memory_space=pl.ANY),
        out_shape=jax.ShapeDtypeStruct((n, d), jnp.float32),
        scratch_shapes=[
            pltpu.VMEM((n, d), jnp.float32),
            pltpu.VMEM((n, d), jnp.float32),
            pltpu.SemaphoreType.DMA((nchunks,)),
            pltpu.SemaphoreType.DMA,
        ],
    )(h, wt, b2)


# speedup vs baseline: 1.0663x; 1.0663x over previous
import functools

import jax
import jax.numpy as jnp
from jax.experimental import pallas as pl
from jax.experimental.pallas import tpu as pltpu


def _pipelined_linear(nchunks, ch, h_hbm, wt_ref, b_ref, o_hbm,
                      inbuf, outbuf, insem, outsem):
    def in_copy(i):
        return pltpu.make_async_copy(
            h_hbm.at[pl.ds(i * ch, ch), :],
            inbuf.at[pl.ds(i * ch, ch), :],
            insem.at[i],
        )

    for i in range(nchunks):
        in_copy(i).start()
    for i in range(nchunks):
        in_copy(i).wait()
        rows = pl.ds(i * ch, ch)
        acc = jnp.dot(inbuf[rows, :], wt_ref[:, :],
                      preferred_element_type=jnp.float32)
        outbuf[rows, :] = acc + b_ref[:, :]
    out = pltpu.make_async_copy(outbuf, o_hbm, outsem)
    out.start()
    out.wait()


def kernel(h, edge_index, edge_sh, edge_radial, n_atoms, W, b):
    n, d = h.shape
    ch = 5000
    nchunks = n // ch if (n % ch == 0) else 1
    if n % ch != 0:
        ch = n
    wt = W.T
    b2 = b.reshape(1, d)
    return pl.pallas_call(
        functools.partial(_pipelined_linear, nchunks, ch),
        in_specs=[
            pl.BlockSpec(memory_space=pl.ANY),
            pl.BlockSpec((d, d), lambda: (0, 0)),
            pl.BlockSpec((1, d), lambda: (0, 0)),
        ],
        out_specs=pl.BlockSpec(memory_space=pl.ANY),
        out_shape=jax.ShapeDtypeStruct((n, d), jnp.float32),
        scratch_shapes=[
            pltpu.VMEM((n, d), jnp.float32),
            pltpu.VMEM((n, d), jnp.float32),
            pltpu.SemaphoreType.DMA((nchunks,)),
            pltpu.SemaphoreType.DMA,
        ],
    )(h, wt, b2)


# speedup vs baseline: 1.2523x; 1.1744x over previous
import functools

import jax
import jax.numpy as jnp
from jax.experimental import pallas as pl
from jax.experimental.pallas import tpu as pltpu


def _pipelined_linear(nchunks, ch, h_hbm, wt_ref, b_ref, o_hbm,
                      inbuf, outbuf, insem, outsem):
    def in_copy(i):
        return pltpu.make_async_copy(
            h_hbm.at[pl.ds(i * ch, ch), :],
            inbuf.at[pl.ds(i * ch, ch), :],
            insem.at[i],
        )

    for i in range(nchunks):
        in_copy(i).start()
    for i in range(nchunks):
        in_copy(i).wait()
        rows = pl.ds(i * ch, ch)
        acc = jnp.dot(inbuf[rows, :], wt_ref[:, :],
                      preferred_element_type=jnp.float32)
        outbuf[rows, :] = acc + b_ref[:, :]
    out = pltpu.make_async_copy(outbuf, o_hbm, outsem)
    out.start()
    out.wait()


def kernel(h, edge_index, edge_sh, edge_radial, n_atoms, W, b):
    n, d = h.shape
    ch = n
    nchunks = n // ch if (n % ch == 0) else 1
    if n % ch != 0:
        ch = n
    wt = W.T
    b2 = b.reshape(1, d)
    return pl.pallas_call(
        functools.partial(_pipelined_linear, nchunks, ch),
        in_specs=[
            pl.BlockSpec(memory_space=pl.ANY),
            pl.BlockSpec((d, d), lambda: (0, 0)),
            pl.BlockSpec((1, d), lambda: (0, 0)),
        ],
        out_specs=pl.BlockSpec(memory_space=pl.ANY),
        out_shape=jax.ShapeDtypeStruct((n, d), jnp.float32),
        scratch_shapes=[
            pltpu.VMEM((n, d), jnp.float32),
            pltpu.VMEM((n, d), jnp.float32),
            pltpu.SemaphoreType.DMA((nchunks,)),
            pltpu.SemaphoreType.DMA,
        ],
    )(h, wt, b2)


# speedup vs baseline: 1.3971x; 1.1156x over previous
import jax
import jax.numpy as jnp
from jax.experimental import pallas as pl
from jax.experimental.pallas import tpu as pltpu


def _linear_kernel(h_ref, wt_ref, b_ref, o_ref):
    acc = jnp.dot(h_ref[:, :], wt_ref[:, :], preferred_element_type=jnp.float32)
    o_ref[:, :] = acc + b_ref[:, :]


def kernel(h, edge_index, edge_sh, edge_radial, n_atoms, W, b):
    n, d = h.shape
    wt = W.T
    b2 = b.reshape(1, d)
    return pl.pallas_call(
        _linear_kernel,
        in_specs=[
            pl.BlockSpec(memory_space=pltpu.VMEM),
            pl.BlockSpec(memory_space=pltpu.VMEM),
            pl.BlockSpec(memory_space=pltpu.VMEM),
        ],
        out_specs=pl.BlockSpec(memory_space=pltpu.VMEM),
        out_shape=jax.ShapeDtypeStruct((n, d), jnp.float32),
    )(h, wt, b2)
